# Initial kernel scaffold; baseline (speedup 1.0000x reference)
import jax
import jax.numpy as jnp
from jax.experimental import pallas as pl


def kernel(logits, target_long):
    raise NotImplementedError("write your pallas kernel here")


# speedup vs baseline: 37.8035x; 37.8035x over previous
import jax
import jax.numpy as jnp
from jax.experimental import pallas as pl
from jax.experimental.pallas import tpu as pltpu

B = 4
C = 4
R = 1024
W = 1024
N = R * W
CHUNK_R = 64
NCHUNK = R // CHUNK_R
K = max(1, int(N * 0.2))


def _body(logits_ref, target_ref, out_ref, keys):
    j = pl.program_id(0)

    @pl.when(j < NCHUNK)
    def _compute_ce():
        x = logits_ref[...]
        t = target_ref[...]
        m = jnp.max(x, axis=1)
        s = jnp.sum(jnp.exp(x - m[:, None]), axis=1)
        lse = m + jnp.log(s)
        xt = x[:, 0]
        for c in range(1, C):
            xt = jnp.where(t == c, x[:, c], xt)
        ce = jnp.maximum(lse - xt, 0.0)
        keys[:, pl.ds(j * CHUNK_R, CHUNK_R), :] = jax.lax.bitcast_convert_type(
            ce, jnp.int32)

    @pl.when(j == NCHUNK)
    def _select():
        def count_ge(mid):
            def chunk(c, acc):
                kk = keys[:, pl.ds(c * CHUNK_R, CHUNK_R), :]
                return acc + jnp.sum((kk >= mid).astype(jnp.int32),
                                     axis=(1, 2), keepdims=True)
            return jax.lax.fori_loop(0, NCHUNK, chunk,
                                     jnp.zeros((B, 1, 1), jnp.int32))

        def bs(i, lohi):
            lo, hi = lohi
            mid = hi - (hi - lo) // 2
            ge = count_ge(mid) >= K
            return jnp.where(ge, mid, lo), jnp.where(ge, hi, mid - 1)

        lo, hi = jax.lax.fori_loop(
            0, 31, bs,
            (jnp.zeros((B, 1, 1), jnp.int32),
             jnp.full((B, 1, 1), 0x7FFFFFFF, jnp.int32)))
        t_key = lo

        def tail(c, acc):
            sm, cnt = acc
            kk = keys[:, pl.ds(c * CHUNK_R, CHUNK_R), :]
            vals = jax.lax.bitcast_convert_type(kk, jnp.float32)
            gt = kk > t_key
            sm = sm + jnp.sum(jnp.where(gt, vals, 0.0),
                              axis=(1, 2), keepdims=True)
            cnt = cnt + jnp.sum(gt.astype(jnp.int32),
                                axis=(1, 2), keepdims=True)
            return sm, cnt

        sm, cnt = jax.lax.fori_loop(
            0, NCHUNK, tail,
            (jnp.zeros((B, 1, 1), jnp.float32),
             jnp.zeros((B, 1, 1), jnp.int32)))
        t_val = jax.lax.bitcast_convert_type(t_key, jnp.float32)
        total = jnp.sum(sm + (K - cnt).astype(jnp.float32) * t_val)
        out_ref[...] = (total / jnp.float32(B * K)).reshape(1, 1)


def kernel(logits, target_long):
    logits_r = logits.reshape(B, C, R, W)
    target_r = target_long.reshape(B, R, W).astype(jnp.int32)
    out = pl.pallas_call(
        _body,
        grid=(NCHUNK + 1,),
        in_specs=[
            pl.BlockSpec((B, C, CHUNK_R, W),
                         lambda j: (0, 0, jnp.minimum(j, NCHUNK - 1), 0)),
            pl.BlockSpec((B, CHUNK_R, W),
                         lambda j: (0, jnp.minimum(j, NCHUNK - 1), 0)),
        ],
        out_specs=pl.BlockSpec((1, 1), lambda j: (0, 0)),
        out_shape=jax.ShapeDtypeStruct((1, 1), jnp.float32),
        scratch_shapes=[pltpu.VMEM((B, R, W), jnp.int32)],
    )(logits_r, target_r)
    return out[0, 0]
